# baseline (speedup 1.0000x reference)
import jax
import jax.numpy as jnp
from jax import lax
from jax.experimental import pallas as pl
from jax.experimental.pallas import tpu as pltpu
from jax.experimental.pallas import tpu_sc as plsc

N_NODES = 100000
D_FEAT = 128
N_SEG = 512
CHUNK = 128
NFULL = N_NODES // CHUNK
TAIL = N_NODES - NFULL * CHUNK
NC = 2
NS = 16
NW = NC * NS
CPW = (NFULL + NW - 1) // NW
NPAIR = (CPW + 1) // 2


def _phase1(x_hbm, b_hbm, part_hbm, cntp_hbm,
            xbA, xbB, ibA, ibB, tidx, cnt_local,
            acc_sh, semxA, semiA, semxB, semiB, semsA, semsB):
    cid = lax.axis_index("c")
    sid = lax.axis_index("s")
    w = sid * NC + cid

    def _init_cnt(i, _):
        cnt_local[i, :] = jnp.zeros((16,), jnp.float32)
        return 0
    lax.fori_loop(0, N_SEG, _init_cnt, 0)

    def _init_z(k, _):
        xbA[k // 8, pl.ds((k % 8) * 16, 16)] = jnp.zeros((16,), jnp.float32)
        return 0
    lax.fori_loop(0, 32 * 8, _init_z, 0)
    pltpu.sync_copy(xbA.at[pl.ds(0, 32)], acc_sh.at[pl.ds(sid * 32, 32)])

    def start_in(g, xb, ib, semx, semi):
        row0 = g * CHUNK
        pltpu.async_copy(b_hbm.at[pl.ds(row0, CHUNK)], ib, semi)
        pltpu.async_copy(x_hbm.at[pl.ds(row0, CHUNK)], xb, semx)

    def wait_in(xb, ib, semx, semi):
        pltpu.make_async_copy(b_hbm.at[pl.ds(0, CHUNK)], ib, semi).wait()
        pltpu.make_async_copy(x_hbm.at[pl.ds(0, CHUNK)], xb, semx).wait()

    ones16 = jnp.ones((16,), jnp.float32)

    def count_chunk(ib, ngroups):
        def body(j, _):
            v = ib[pl.ds(j * 16, 16)]
            for k in range(16):
                plsc.addupdate(cnt_local.at[v[k]], ones16)
            return 0
        lax.fori_loop(0, ngroups, body, 0)

    start_in(w, xbA, ibA, semxA, semiA)
    plsc.subcore_barrier()

    def _pair(c2, _):
        ge = w + NW * (2 * c2)
        go = ge + NW
        gne = ge + 2 * NW

        @pl.when(go < NFULL)
        def _():
            start_in(go, xbB, ibB, semxB, semiB)

        @pl.when(ge < NFULL)
        def _():
            wait_in(xbA, ibA, semxA, semiA)
            pltpu.async_copy(xbA, acc_sh.at[ibA], semsA, add=True)
            count_chunk(ibA, CHUNK // 16)

        @pl.when(go < NFULL)
        def _():
            wait_in(xbB, ibB, semxB, semiB)
            pltpu.async_copy(xbB, acc_sh.at[ibB], semsB, add=True)
            count_chunk(ibB, CHUNK // 16)

        @pl.when(ge < NFULL)
        def _():
            pltpu.make_async_copy(xbA, acc_sh.at[ibA], semsA).wait()

        @pl.when(gne < NFULL)
        def _():
            start_in(gne, xbA, ibA, semxA, semiA)

        @pl.when(go < NFULL)
        def _():
            pltpu.make_async_copy(xbB, acc_sh.at[ibB], semsB).wait()
        return 0

    lax.fori_loop(0, NPAIR, _pair, 0)

    @pl.when(w == (NFULL % NW))
    def _():
        row0 = NFULL * CHUNK
        pltpu.async_copy(b_hbm.at[pl.ds(row0, TAIL)], tidx, semiA).wait()
        pltpu.async_copy(x_hbm.at[pl.ds(row0, TAIL)],
                         xbA.at[pl.ds(0, TAIL)], semxA).wait()
        pltpu.async_copy(xbA.at[pl.ds(0, TAIL)], acc_sh.at[tidx],
                         semsA, add=True)
        count_chunk(tidx, TAIL // 16)
        pltpu.make_async_copy(xbA.at[pl.ds(0, TAIL)], acc_sh.at[tidx],
                              semsA).wait()

    pltpu.sync_copy(cnt_local, cntp_hbm.at[w])

    plsc.subcore_barrier()

    @pl.when(sid == 0)
    def _():
        pltpu.sync_copy(acc_sh, part_hbm.at[cid])


def _combine(p_ref, c_ref, o_ref):
    s = p_ref[0] + p_ref[1]
    cnt = jnp.sum(c_ref[...], axis=0)[:, 0:1]
    o_ref[...] = s / jnp.maximum(cnt, 1.0)


@jax.jit
def kernel(x, batch):
    mesh = plsc.VectorSubcoreMesh(core_axis_name="c", subcore_axis_name="s")
    phase1 = pl.kernel(
        _phase1,
        out_type=[
            jax.ShapeDtypeStruct((NC, N_SEG, D_FEAT), jnp.float32),
            jax.ShapeDtypeStruct((NW, N_SEG, 16), jnp.float32),
        ],
        mesh=mesh,
        scratch_types=[
            pltpu.VMEM((CHUNK, D_FEAT), jnp.float32),
            pltpu.VMEM((CHUNK, D_FEAT), jnp.float32),
            pltpu.VMEM((CHUNK,), jnp.int32),
            pltpu.VMEM((CHUNK,), jnp.int32),
            pltpu.VMEM((TAIL,), jnp.int32),
            pltpu.VMEM((N_SEG, 16), jnp.float32),
            pltpu.VMEM_SHARED((N_SEG, D_FEAT), jnp.float32),
            pltpu.SemaphoreType.DMA,
            pltpu.SemaphoreType.DMA,
            pltpu.SemaphoreType.DMA,
            pltpu.SemaphoreType.DMA,
            pltpu.SemaphoreType.DMA,
            pltpu.SemaphoreType.DMA,
        ],
    )
    partials, cnts = phase1(x, batch)
    out = pl.pallas_call(
        _combine,
        out_shape=jax.ShapeDtypeStruct((N_SEG, D_FEAT), jnp.float32),
    )(partials, cnts)
    return out


# speedup vs baseline: 1.1478x; 1.1478x over previous
import jax
import jax.numpy as jnp
from jax import lax
from jax.experimental import pallas as pl
from jax.experimental.pallas import tpu as pltpu
from jax.experimental.pallas import tpu_sc as plsc

N_NODES = 100000
D_FEAT = 128
N_SEG = 512
CHUNK = 128
NFULL = N_NODES // CHUNK
TAIL = N_NODES - NFULL * CHUNK
NC = 2
NS = 16
NW = NC * NS
CPW = (NFULL + NW - 1) // NW
NPAIR = (CPW + 1) // 2


def _phase1(x_hbm, b_hbm, part_hbm, cntp_hbm,
            xbA, xbB, ibA, ibB, tidx, cnt_local,
            acc_sh, semxA, semiA, semxB, semiB, semsA, semsB):
    cid = lax.axis_index("c")
    sid = lax.axis_index("s")
    w = sid * NC + cid

    def _init_cnt(i, _):
        cnt_local[i, :] = jnp.zeros((16,), jnp.float32)
        return 0
    lax.fori_loop(0, N_SEG, _init_cnt, 0)

    def _init_z(k, _):
        xbA[k // 8, pl.ds((k % 8) * 16, 16)] = jnp.zeros((16,), jnp.float32)
        return 0
    lax.fori_loop(0, 32 * 8, _init_z, 0)
    pltpu.sync_copy(xbA.at[pl.ds(0, 32)], acc_sh.at[pl.ds(sid * 32, 32)])

    def start_in(g, xb, ib, semx, semi):
        row0 = g * CHUNK
        pltpu.async_copy(b_hbm.at[pl.ds(row0, CHUNK)], ib, semi)
        pltpu.async_copy(x_hbm.at[pl.ds(row0, CHUNK)], xb, semx)

    def wait_in(xb, ib, semx, semi):
        pltpu.make_async_copy(b_hbm.at[pl.ds(0, CHUNK)], ib, semi).wait()
        pltpu.make_async_copy(x_hbm.at[pl.ds(0, CHUNK)], xb, semx).wait()

    ones16 = jnp.ones((16,), jnp.float32)

    def count_chunk(ib, ngroups):
        def body(j, _):
            v = ib[pl.ds(j * 16, 16)]
            for k in range(16):
                plsc.addupdate(cnt_local.at[v[k]], ones16)
            return 0
        lax.fori_loop(0, ngroups, body, 0)

    start_in(w, xbA, ibA, semxA, semiA)
    plsc.subcore_barrier()

    def _pair(c2, _):
        ge = w + NW * (2 * c2)
        go = ge + NW
        gne = ge + 2 * NW

        @pl.when(go < NFULL)
        def _():
            start_in(go, xbB, ibB, semxB, semiB)

        @pl.when(ge < NFULL)
        def _():
            wait_in(xbA, ibA, semxA, semiA)
            pltpu.async_copy(xbA, acc_sh.at[ibA], semsA, add=True)
            count_chunk(ibA, CHUNK // 16)
            pltpu.make_async_copy(xbA, acc_sh.at[ibA], semsA).wait()

        @pl.when(gne < NFULL)
        def _():
            start_in(gne, xbA, ibA, semxA, semiA)

        @pl.when(go < NFULL)
        def _():
            wait_in(xbB, ibB, semxB, semiB)
            pltpu.async_copy(xbB, acc_sh.at[ibB], semsB, add=True)
            count_chunk(ibB, CHUNK // 16)
            pltpu.make_async_copy(xbB, acc_sh.at[ibB], semsB).wait()
        return 0

    lax.fori_loop(0, NPAIR, _pair, 0)

    @pl.when(w == (NFULL % NW))
    def _():
        row0 = NFULL * CHUNK
        pltpu.async_copy(b_hbm.at[pl.ds(row0, TAIL)], tidx, semiA).wait()
        pltpu.async_copy(x_hbm.at[pl.ds(row0, TAIL)],
                         xbA.at[pl.ds(0, TAIL)], semxA).wait()
        pltpu.async_copy(xbA.at[pl.ds(0, TAIL)], acc_sh.at[tidx],
                         semsA, add=True)
        count_chunk(tidx, TAIL // 16)
        pltpu.make_async_copy(xbA.at[pl.ds(0, TAIL)], acc_sh.at[tidx],
                              semsA).wait()

    pltpu.sync_copy(cnt_local, cntp_hbm.at[w])

    plsc.subcore_barrier()

    @pl.when(sid == 0)
    def _():
        pltpu.sync_copy(acc_sh, part_hbm.at[cid])


def _combine(p_ref, c_ref, o_ref):
    s = p_ref[0] + p_ref[1]
    cnt = jnp.sum(c_ref[...], axis=0)[:, 0:1]
    o_ref[...] = s / jnp.maximum(cnt, 1.0)


@jax.jit
def kernel(x, batch):
    mesh = plsc.VectorSubcoreMesh(core_axis_name="c", subcore_axis_name="s")
    phase1 = pl.kernel(
        _phase1,
        out_type=[
            jax.ShapeDtypeStruct((NC, N_SEG, D_FEAT), jnp.float32),
            jax.ShapeDtypeStruct((NW, N_SEG, 16), jnp.float32),
        ],
        mesh=mesh,
        scratch_types=[
            pltpu.VMEM((CHUNK, D_FEAT), jnp.float32),
            pltpu.VMEM((CHUNK, D_FEAT), jnp.float32),
            pltpu.VMEM((CHUNK,), jnp.int32),
            pltpu.VMEM((CHUNK,), jnp.int32),
            pltpu.VMEM((TAIL,), jnp.int32),
            pltpu.VMEM((N_SEG, 16), jnp.float32),
            pltpu.VMEM_SHARED((N_SEG, D_FEAT), jnp.float32),
            pltpu.SemaphoreType.DMA,
            pltpu.SemaphoreType.DMA,
            pltpu.SemaphoreType.DMA,
            pltpu.SemaphoreType.DMA,
            pltpu.SemaphoreType.DMA,
            pltpu.SemaphoreType.DMA,
        ],
    )
    partials, cnts = phase1(x, batch)
    out = pl.pallas_call(
        _combine,
        out_shape=jax.ShapeDtypeStruct((N_SEG, D_FEAT), jnp.float32),
    )(partials, cnts)
    return out
